# baseline (speedup 1.0000x reference)
import jax
import jax.numpy as jnp
from jax.experimental import pallas as pl
from jax.experimental.pallas import tpu as pltpu

_B = 512
_KV = 512

_ROT = ((13, 15, 26, 6), (17, 29, 16, 24))
_KS = (0, 42, 0x1BD11BF0)


def _gumbel_tile(base, shape, k):
    row = jax.lax.broadcasted_iota(jnp.uint32, shape, 0)
    col = jax.lax.broadcasted_iota(jnp.uint32, shape, 1)
    cnt = row * jnp.uint32(k) + col + base
    x0 = jnp.zeros(shape, jnp.uint32) + jnp.uint32(_KS[0])
    x1 = cnt + jnp.uint32(_KS[1])
    for group in range(5):
        for r in _ROT[group % 2]:
            x0 = x0 + x1
            x1 = (x1 << jnp.uint32(r)) | (x1 >> jnp.uint32(32 - r))
            x1 = x1 ^ x0
        x0 = x0 + jnp.uint32(_KS[(group + 1) % 3])
        x1 = x1 + jnp.uint32((_KS[(group + 2) % 3] + group + 1) & 0xFFFFFFFF)
    bits = x0 ^ x1
    fb = (bits >> jnp.uint32(9)) | jnp.uint32(0x3F800000)
    fl = jax.lax.bitcast_convert_type(fb, jnp.float32) - jnp.float32(1.0)
    mn = jnp.float32(1e-6)
    mx = jnp.float32(1.0 - 1e-6)
    u = jnp.maximum(mn, fl * (mx - mn) + mn)
    return -jnp.log(-jnp.log(u))


def _gumbel_const(shape, dtype):
    u = jax.random.uniform(jax.random.key(42), shape,
                           minval=1e-6, maxval=1.0 - 1e-6, dtype=dtype)
    return -jnp.log(-jnp.log(u))


def _vq_body(x_ref, g_ref, t_ref, cb_ref, emb_ref, ids_ref):
    i = pl.program_id(0)
    xb = x_ref[...]
    cb = cb_ref[...]
    k = cb.shape[0]
    s = jax.lax.dot_general(xb, cb, (((1,), (1,)), ((), ())),
                            preferred_element_type=jnp.float32)
    x2 = jnp.sum(xb * xb, axis=1, keepdims=True)
    c2 = jnp.sum(cb * cb, axis=1)[None, :]
    dist = (x2 + c2) - 2.0 * s
    mn = jnp.min(dist, axis=1, keepdims=True)
    iota = jax.lax.broadcasted_iota(jnp.int32, dist.shape, 1)
    ids_ref[...] = jnp.min(jnp.where(dist == mn, iota, k), axis=1,
                           keepdims=True)
    inv_t = 1.0 / t_ref[0]
    base = (i * _B * k).astype(jnp.uint32)
    gv = _gumbel_tile(base, (dist.shape[0], _KV), k)
    g = jnp.concatenate([gv, g_ref[...].astype(jnp.float32)], axis=1)
    z = g - dist
    m = jnp.max(z, axis=1, keepdims=True)
    e = jnp.exp((z - m) * inv_t)
    w = e / jnp.sum(e, axis=1, keepdims=True)
    emb_ref[...] = jax.lax.dot_general(w, cb, (((1,), (0,)), ((), ())),
                                       preferred_element_type=jnp.float32)


def kernel(x, temperature, codebook):
    n, d = x.shape
    k = codebook.shape[0]
    t1 = jnp.asarray(temperature, jnp.float32).reshape(1)
    gd = jnp.copy(_gumbel_const((n, k), jnp.float32)[:, _KV:]).astype(jnp.bfloat16)
    emb, ids2 = pl.pallas_call(
        _vq_body,
        grid=(n // _B,),
        in_specs=[
            pl.BlockSpec((_B, d), lambda i: (i, 0)),
            pl.BlockSpec((_B, k - _KV), lambda i: (i, 0)),
            pl.BlockSpec(memory_space=pltpu.SMEM),
            pl.BlockSpec((k, d), lambda i: (0, 0)),
        ],
        out_specs=[
            pl.BlockSpec((_B, d), lambda i: (i, 0)),
            pl.BlockSpec((_B, 1), lambda i: (i, 0)),
        ],
        out_shape=[
            jax.ShapeDtypeStruct((n, d), jnp.float32),
            jax.ShapeDtypeStruct((n, 1), jnp.int32),
        ],
        compiler_params=pltpu.CompilerParams(
            dimension_semantics=("arbitrary",)),
    )(x, gd, t1, codebook)
    return emb, ids2[:, 0]


# speedup vs baseline: 1.0001x; 1.0001x over previous
import jax
import jax.numpy as jnp
from jax.experimental import pallas as pl
from jax.experimental.pallas import tpu as pltpu

_B = 512
_KV = 512

_ROT = ((13, 15, 26, 6), (17, 29, 16, 24))
_KS = (0, 42, 0x1BD11BF0)


def _gumbel_tile(base, shape, k):
    kshift = k.bit_length() - 1
    row = jax.lax.broadcasted_iota(jnp.uint32, shape, 0)
    col = jax.lax.broadcasted_iota(jnp.uint32, shape, 1)
    cnt = ((row << jnp.uint32(kshift)) | col) + base
    x0 = jnp.zeros(shape, jnp.uint32) + jnp.uint32(_KS[0])
    x1 = cnt + jnp.uint32(_KS[1])
    for group in range(5):
        for r in _ROT[group % 2]:
            x0 = x0 + x1
            x1 = (x1 << jnp.uint32(r)) | (x1 >> jnp.uint32(32 - r))
            x1 = x1 ^ x0
        x0 = x0 + jnp.uint32(_KS[(group + 1) % 3])
        x1 = x1 + jnp.uint32((_KS[(group + 2) % 3] + group + 1) & 0xFFFFFFFF)
    bits = x0 ^ x1
    fb = (bits >> jnp.uint32(9)) | jnp.uint32(0x3F800000)
    fl = jax.lax.bitcast_convert_type(fb, jnp.float32) - jnp.float32(1.0)
    mn = jnp.float32(1e-6)
    mx = jnp.float32(1.0 - 1e-6)
    u = jnp.maximum(mn, fl * (mx - mn) + mn)
    return -jnp.log(-jnp.log(u))


def _gumbel_const(shape, dtype):
    u = jax.random.uniform(jax.random.key(42), shape,
                           minval=1e-6, maxval=1.0 - 1e-6, dtype=dtype)
    return -jnp.log(-jnp.log(u))


def _vq_body(x_ref, g_ref, t_ref, cb_ref, emb_ref, ids_ref):
    i = pl.program_id(0)
    xb = x_ref[...]
    cb = cb_ref[...]
    k = cb.shape[0]
    s = jax.lax.dot_general(xb, cb, (((1,), (1,)), ((), ())),
                            preferred_element_type=jnp.float32)
    x2 = jnp.sum(xb * xb, axis=1, keepdims=True)
    c2 = jnp.sum(cb * cb, axis=1)[None, :]
    dist = (x2 + c2) - 2.0 * s
    mn = jnp.min(dist, axis=1, keepdims=True)
    iota = jax.lax.broadcasted_iota(jnp.int32, dist.shape, 1)
    ids_ref[...] = jnp.min(jnp.where(dist == mn, iota, k), axis=1,
                           keepdims=True)
    inv_t = 1.0 / t_ref[0]
    base = (i * _B * k).astype(jnp.uint32)
    gl = _gumbel_tile(base, (xb.shape[0], _KV), k)
    zl = gl - dist[:, :_KV]
    zr = g_ref[...].astype(jnp.float32) - dist[:, _KV:]
    m = jnp.maximum(jnp.max(zl, axis=1, keepdims=True),
                    jnp.max(zr, axis=1, keepdims=True))
    el = jnp.exp((zl - m) * inv_t)
    er = jnp.exp((zr - m) * inv_t)
    rs = 1.0 / (jnp.sum(el, axis=1, keepdims=True)
                + jnp.sum(er, axis=1, keepdims=True))
    emb_ref[...] = (
        jax.lax.dot_general(el * rs, cb[:_KV], (((1,), (0,)), ((), ())),
                            preferred_element_type=jnp.float32)
        + jax.lax.dot_general(er * rs, cb[_KV:], (((1,), (0,)), ((), ())),
                              preferred_element_type=jnp.float32))


def kernel(x, temperature, codebook):
    n, d = x.shape
    k = codebook.shape[0]
    t1 = jnp.asarray(temperature, jnp.float32).reshape(1)
    gd = _gumbel_const((n, k), jnp.float32).astype(jnp.bfloat16)
    emb, ids2 = pl.pallas_call(
        _vq_body,
        grid=(n // _B,),
        in_specs=[
            pl.BlockSpec((_B, d), lambda i: (i, 0)),
            pl.BlockSpec((_B, k - _KV), lambda i: (i, 1)),
            pl.BlockSpec(memory_space=pltpu.SMEM),
            pl.BlockSpec((k, d), lambda i: (0, 0)),
        ],
        out_specs=[
            pl.BlockSpec((_B, d), lambda i: (i, 0)),
            pl.BlockSpec((_B, 1), lambda i: (i, 0)),
        ],
        out_shape=[
            jax.ShapeDtypeStruct((n, d), jnp.float32),
            jax.ShapeDtypeStruct((n, 1), jnp.int32),
        ],
        compiler_params=pltpu.CompilerParams(
            dimension_semantics=("arbitrary",)),
    )(x, gd, t1, codebook)
    return emb, ids2[:, 0]


# speedup vs baseline: 1.4737x; 1.4736x over previous
import jax
import jax.numpy as jnp
from jax.experimental import pallas as pl
from jax.experimental.pallas import tpu as pltpu

_B = 1024

_ROT = ((13, 15, 26, 6), (17, 29, 16, 24))
_KS = (0, 42, 0x1BD11BF0)
_NEG_LN2 = -0.6931471805599453


def _gumbel_tile(base, shape, k):
    kshift = k.bit_length() - 1
    row = jax.lax.broadcasted_iota(jnp.uint32, shape, 0)
    col = jax.lax.broadcasted_iota(jnp.uint32, shape, 1)
    cnt = ((row << jnp.uint32(kshift)) | col) + base
    x1 = cnt + jnp.uint32(_KS[1])
    x0 = x1
    first = True
    for group in range(5):
        for r in _ROT[group % 2]:
            if first:
                first = False
            else:
                x0 = x0 + x1
            x1 = (x1 << jnp.uint32(r)) | (x1 >> jnp.uint32(32 - r))
            x1 = x1 ^ x0
        x0 = x0 + jnp.uint32(_KS[(group + 1) % 3])
        x1 = x1 + jnp.uint32((_KS[(group + 2) % 3] + group + 1) & 0xFFFFFFFF)
    bits = x0 ^ x1
    fb = (bits >> jnp.uint32(9)) | jnp.uint32(0x3F800000)
    fl = jax.lax.bitcast_convert_type(fb, jnp.float32) - jnp.float32(1.0)
    mn = jnp.float32(1e-6)
    mx = jnp.float32(1.0 - 1e-6)
    u = fl * (mx - mn) + mn
    nlog_u = jnp.log2(u) * jnp.float32(_NEG_LN2)
    return jnp.log2(nlog_u) * jnp.float32(_NEG_LN2)


def _vq_body(x_ref, t_ref, cb_ref, emb_ref, ids_ref):
    i = pl.program_id(0)
    xb = x_ref[...]
    cb = cb_ref[...]
    k = cb.shape[0]
    s = jax.lax.dot_general(xb, cb, (((1,), (1,)), ((), ())),
                            preferred_element_type=jnp.float32)
    x2 = jnp.sum(xb * xb, axis=1, keepdims=True)
    c2 = jnp.sum(cb * cb, axis=1)[None, :]
    dist = (x2 + c2) - 2.0 * s
    mn = jnp.min(dist, axis=1, keepdims=True)
    iota = jax.lax.broadcasted_iota(jnp.int32, dist.shape, 1)
    ids_ref[...] = jnp.min(jnp.where(dist == mn, iota, k), axis=1,
                           keepdims=True)
    inv_t = 1.0 / t_ref[0]
    base = (i * _B * k).astype(jnp.uint32)
    g = _gumbel_tile(base, dist.shape, k)
    z = g - dist
    m = jnp.max(z, axis=1, keepdims=True)
    e = jnp.exp((z - m) * inv_t)
    w = e / jnp.sum(e, axis=1, keepdims=True)
    emb_ref[...] = jax.lax.dot_general(w, cb, (((1,), (0,)), ((), ())),
                                       preferred_element_type=jnp.float32)


def kernel(x, temperature, codebook):
    n, d = x.shape
    k = codebook.shape[0]
    t1 = jnp.asarray(temperature, jnp.float32).reshape(1)
    emb, ids2 = pl.pallas_call(
        _vq_body,
        grid=(n // _B,),
        in_specs=[
            pl.BlockSpec((_B, d), lambda i: (i, 0)),
            pl.BlockSpec(memory_space=pltpu.SMEM),
            pl.BlockSpec((k, d), lambda i: (0, 0)),
        ],
        out_specs=[
            pl.BlockSpec((_B, d), lambda i: (i, 0)),
            pl.BlockSpec((_B, 1), lambda i: (i, 0)),
        ],
        out_shape=[
            jax.ShapeDtypeStruct((n, d), jnp.float32),
            jax.ShapeDtypeStruct((n, 1), jnp.int32),
        ],
        compiler_params=pltpu.CompilerParams(
            dimension_semantics=("arbitrary",)),
    )(x, t1, codebook)
    return emb, ids2[:, 0]


# speedup vs baseline: 1.4766x; 1.0019x over previous
import jax
import jax.numpy as jnp
from jax.experimental import pallas as pl
from jax.experimental.pallas import tpu as pltpu

_B = 1024

_ROT = ((13, 15, 26, 6), (17, 29, 16, 24))
_KS = (0, 42, 0x1BD11BF0)
_NEG_LN2 = -0.6931471805599453


def _gumbel_tile(base, shape, k):
    kshift = k.bit_length() - 1
    row = jax.lax.broadcasted_iota(jnp.uint32, shape, 0)
    col = jax.lax.broadcasted_iota(jnp.uint32, shape, 1)
    cnt = ((row << jnp.uint32(kshift)) | col) + base
    x1 = cnt + jnp.uint32(_KS[1])
    x0 = x1
    first = True
    for group in range(5):
        for r in _ROT[group % 2]:
            if first:
                first = False
            else:
                x0 = x0 + x1
            x1 = (x1 << jnp.uint32(r)) | (x1 >> jnp.uint32(32 - r))
            x1 = x1 ^ x0
        x0 = x0 + jnp.uint32(_KS[(group + 1) % 3])
        x1 = x1 + jnp.uint32((_KS[(group + 2) % 3] + group + 1) & 0xFFFFFFFF)
    bits = x0 ^ x1
    fb = (bits >> jnp.uint32(9)) | jnp.uint32(0x3F800000)
    fl = jax.lax.bitcast_convert_type(fb, jnp.float32) - jnp.float32(1.0)
    mn = jnp.float32(1e-6)
    mx = jnp.float32(1.0 - 1e-6)
    u = fl * (mx - mn) + mn
    nlog_u = jnp.log2(u) * jnp.float32(_NEG_LN2)
    return jnp.log2(nlog_u) * jnp.float32(_NEG_LN2)


def _vq_body(x_ref, t_ref, cb_ref, emb_ref, ids_ref):
    i = pl.program_id(0)
    xb = x_ref[...]
    cb = cb_ref[...]
    k = cb.shape[0]
    s = jax.lax.dot_general(xb, cb, (((1,), (1,)), ((), ())),
                            preferred_element_type=jnp.float32)
    x2 = jnp.sum(xb * xb, axis=1, keepdims=True)
    c2 = jnp.sum(cb * cb, axis=1)[None, :]
    dist = (x2 + c2) - 2.0 * s
    mn = jnp.min(dist, axis=1, keepdims=True)
    iota = jax.lax.broadcasted_iota(jnp.int32, dist.shape, 1)
    ids_ref[...] = jnp.min(jnp.where(dist == mn, iota, k), axis=1,
                           keepdims=True)
    inv_t = 1.0 / t_ref[0]
    base = (i * _B * k).astype(jnp.uint32)
    g = _gumbel_tile(base, dist.shape, k)
    z = g - dist
    m = jnp.max(z, axis=1, keepdims=True)
    e = jnp.exp((z - m) * inv_t)
    w = e / jnp.sum(e, axis=1, keepdims=True)
    emb_ref[...] = jax.lax.dot_general(w, cb, (((1,), (0,)), ((), ())),
                                       preferred_element_type=jnp.float32)


def kernel(x, temperature, codebook):
    n, d = x.shape
    k = codebook.shape[0]
    t1 = jnp.asarray(temperature, jnp.float32).reshape(1)
    emb, ids2 = pl.pallas_call(
        _vq_body,
        grid=(n // _B,),
        in_specs=[
            pl.BlockSpec((_B, d), lambda i: (i, 0)),
            pl.BlockSpec(memory_space=pltpu.SMEM),
            pl.BlockSpec((k, d), lambda i: (0, 0)),
        ],
        out_specs=[
            pl.BlockSpec((_B, d), lambda i: (i, 0)),
            pl.BlockSpec((_B, 1), lambda i: (i, 0)),
        ],
        out_shape=[
            jax.ShapeDtypeStruct((n, d), jnp.float32),
            jax.ShapeDtypeStruct((n, 1), jnp.int32),
        ],
        compiler_params=pltpu.CompilerParams(
            dimension_semantics=("parallel",)),
    )(x, t1, codebook)
    return emb, ids2[:, 0]
